# Initial kernel scaffold; baseline (speedup 1.0000x reference)
import jax
import jax.numpy as jnp
from jax.experimental import pallas as pl


def kernel(z, W, b, codebook):
    raise NotImplementedError("write your pallas kernel here")


# speedup vs baseline: 1.0905x; 1.0905x over previous
import functools

import jax
import jax.numpy as jnp
from jax import lax
from jax.experimental import pallas as pl
from jax.experimental.pallas import tpu as pltpu
from jax.experimental.pallas import tpu_sc as plsc

SRC_DIM = 1024
K = 512
IN_DIM = 256
COMMIT = 0.25

BN = 512
NW = 32
CHUNK = 64


def _tc_body(zf_ref, w_ref, b_ref, ct_ref, idx_ref, loss_ref, acc_ref):
    i = pl.program_id(0)
    zp = jnp.dot(zf_ref[...], w_ref[...], preferred_element_type=jnp.float32)
    zp = zp + b_ref[...]
    g = jnp.dot(zp, ct_ref[...], preferred_element_type=jnp.float32)
    rn = jnp.sum(zp * zp, axis=1, keepdims=True)
    cn = jnp.sum(ct_ref[...] * ct_ref[...], axis=0, keepdims=True)
    d = rn + cn - 2.0 * g
    minval = jnp.min(d, axis=1, keepdims=True)
    colid = lax.broadcasted_iota(jnp.int32, d.shape, 1)
    cand = jnp.where(d == minval, colid, K)
    idx_ref[0, 0, :] = jnp.min(cand, axis=1).astype(jnp.int32)

    @pl.when(i == 0)
    def _init():
        acc_ref[0] = 0.0

    acc_ref[0] += jnp.sum(minval)

    @pl.when(i == pl.num_programs(0) - 1)
    def _fini():
        n_total = pl.num_programs(0) * BN
        scale = (1.0 + COMMIT) / (n_total * SRC_DIM)
        loss_ref[...] = jnp.full((1, 1), acc_ref[0] * scale, jnp.float32)


def _tc_distances(zf, W, b2, ct):
    n = zf.shape[0]
    nb = n // BN
    idx3, loss = pl.pallas_call(
        _tc_body,
        grid=(nb,),
        in_specs=[
            pl.BlockSpec((BN, IN_DIM), lambda i: (i, 0)),
            pl.BlockSpec((IN_DIM, SRC_DIM), lambda i: (0, 0)),
            pl.BlockSpec((1, SRC_DIM), lambda i: (0, 0)),
            pl.BlockSpec((SRC_DIM, K), lambda i: (0, 0)),
        ],
        out_specs=[
            pl.BlockSpec((1, 1, BN), lambda i: (i, 0, 0)),
            pl.BlockSpec((1, 1), lambda i: (0, 0)),
        ],
        out_shape=[
            jax.ShapeDtypeStruct((nb, 1, BN), jnp.int32),
            jax.ShapeDtypeStruct((1, 1), jnp.float32),
        ],
        scratch_shapes=[pltpu.SMEM((1,), jnp.float32)],
    )(zf, W, b2, ct)
    return idx3, loss


def _make_sc_gather(n):
    rows_per_w = n // NW
    mesh = plsc.VectorSubcoreMesh(core_axis_name="c", subcore_axis_name="s")

    @functools.partial(
        pl.kernel,
        mesh=mesh,
        out_type=jax.ShapeDtypeStruct((n, SRC_DIM), jnp.float32),
        scratch_types=[
            pltpu.VMEM((CHUNK,), jnp.int32),
            pltpu.VMEM((CHUNK, SRC_DIM), jnp.float32),
            pltpu.SemaphoreType.DMA,
        ],
    )
    def _sc_gather(codebook_hbm, idx_hbm, out_hbm, idx_v, rows_v, sem):
        wid = lax.axis_index("s") * 2 + lax.axis_index("c")
        base = wid * rows_per_w
        for c in range(rows_per_w // CHUNK):
            off = base + c * CHUNK
            pltpu.sync_copy(idx_hbm.at[pl.ds(off, CHUNK)], idx_v)
            pltpu.async_copy(codebook_hbm.at[idx_v], rows_v, sem).wait()
            pltpu.sync_copy(rows_v, out_hbm.at[pl.ds(off, CHUNK)])

    return _sc_gather


def kernel(z, W, b, codebook):
    lead_shape = z.shape[:-1]
    zf = z.reshape(-1, IN_DIM)
    n = zf.shape[0]
    idx3, loss = _tc_distances(zf, W, b.reshape(1, SRC_DIM), codebook.T)
    idx = idx3.reshape(n)
    q = _make_sc_gather(n)(codebook, idx)
    return (
        q.reshape(lead_shape + (SRC_DIM,)),
        loss[0, 0],
        idx[:, None],
    )
